# baseline (speedup 1.0000x reference)
import functools

import jax
import jax.numpy as jnp
from jax import lax
from jax.experimental import pallas as pl
from jax.experimental.pallas import tpu as pltpu
from jax.experimental.pallas import tpu_sc as plsc

B, L, D, V = 4096, 50, 128, 1000
LP = 56
NP = B * LP
NC, NS = 2, 16
NW = NC * NS
BPW = NP // NW
CHUNK = 112
NCHUNK = BPW // CHUNK

TOKB = 1792
BB = TOKB // LP
GRID = NP // TOKB


DH = D // 2
DHP = DH + 1
DP = D + 1


def _sc_gather(tbl_i32, idx):
    mesh = plsc.VectorSubcoreMesh(core_axis_name="c", subcore_axis_name="s")

    @functools.partial(
        pl.kernel,
        mesh=mesh,
        out_type=jax.ShapeDtypeStruct((NP, D), jnp.float32),
        compiler_params=pltpu.CompilerParams(needs_layout_passes=False),
        scratch_types=[
            pltpu.VMEM((V * DHP,), jnp.int32),
            pltpu.VMEM((BPW,), jnp.int32),
            pltpu.VMEM((2, CHUNK, DP), jnp.float32),
            pltpu.SemaphoreType.DMA,
            pltpu.SemaphoreType.DMA,
        ],
    )
    def gather_kernel(tbl_hbm, idx_hbm, out_hbm, tbl_v, idx_all, stage, s0, s1):
        sem_w = (s0, s1)
        wid = lax.axis_index("s") * NC + lax.axis_index("c")
        base = wid * BPW
        pltpu.sync_copy(tbl_hbm, tbl_v)
        pltpu.sync_copy(idx_hbm.at[pl.ds(base, BPW)], idx_all)
        lanes = lax.iota(jnp.int32, 16)

        def w_copy(c, b):
            off = pl.multiple_of(base + c * CHUNK, 8)
            return pltpu.make_async_copy(
                stage.at[b, :, pl.ds(0, D)], out_hbm.at[pl.ds(off, CHUNK)],
                sem_w[b])

        def chunk(c, b):
            @pl.when(c >= 2)
            def _():
                w_copy(c - 2, b).wait()

            def group(tg, carry):
                iv = idx_all[pl.ds(c * CHUNK + tg * 16, 16)]
                ivb = iv * DHP
                rows = tg * 16 + lanes

                @plsc.parallel_loop(0, DH, unroll=32)
                def dbody(d2):
                    w = plsc.load_gather(tbl_v, [ivb + d2])
                    lo = plsc.bitcast(jnp.left_shift(w, 16), jnp.float32)
                    hi = plsc.bitcast(jnp.bitwise_and(w, jnp.int32(-65536)),
                                      jnp.float32)
                    cl = jnp.broadcast_to(d2 * 2, (16,)).astype(jnp.int32)
                    plsc.store_scatter(stage.at[b], [rows, cl], lo)
                    plsc.store_scatter(stage.at[b], [rows, cl + 1], hi)

                return carry

            lax.fori_loop(0, CHUNK // 16, group, 0)
            w_copy(c, b).start()

        def outer(c2, carry):
            for b in range(2):
                chunk(c2 * 2 + b, b)
            return carry

        lax.fori_loop(0, NCHUNK // 2, outer, 0)
        w_copy(NCHUNK - 2, 0).wait()
        w_copy(NCHUNK - 1, 1).wait()

    return gather_kernel(tbl_i32, idx)


def _tc_body(t_ref, v_ref, g_ref, tw1_ref, tb1_ref, tw2t_ref, vw1_ref,
             vw2t_ref, eye_ref, out_ref):
    tb = jnp.broadcast_to(t_ref[0], (D, TOKB))
    vb = jnp.broadcast_to(v_ref[0], (D, TOKB))
    at = jnp.tanh(tw1_ref[...] * tb + tb1_ref[...])
    bt = jnp.tanh(vw1_ref[...] * vb)
    st = lax.dot_general(tw2t_ref[...], at.astype(jnp.bfloat16),
                         (((1,), (0,)), ((), ())),
                         preferred_element_type=jnp.float32)
    st = st + lax.dot_general(vw2t_ref[...], bt.astype(jnp.bfloat16),
                              (((1,), (0,)), ((), ())),
                              preferred_element_type=jnp.float32)
    s = lax.dot_general(st.astype(jnp.bfloat16), eye_ref[...],
                        (((0,), (0,)), ((), ())),
                        preferred_element_type=jnp.float32)
    r = jnp.reshape(g_ref[...] + s, (BB, LP, D))
    out_ref[...] = r[:, :L, :]


def _tc_dense_add(t, v, g, tw1c, tb1c, tw2t, vw1c, vw2t, eye):
    wspec = pl.BlockSpec((D, 1), lambda i: (0, 0))
    mspec = pl.BlockSpec((D, D), lambda i: (0, 0))
    return pl.pallas_call(
        _tc_body,
        grid=(GRID,),
        in_specs=[
            pl.BlockSpec((1, 1, TOKB), lambda i: (i, 0, 0)),
            pl.BlockSpec((1, 1, TOKB), lambda i: (i, 0, 0)),
            pl.BlockSpec((TOKB, D), lambda i: (i, 0)),
            wspec, wspec, mspec, wspec, mspec, mspec,
        ],
        out_specs=pl.BlockSpec((BB, L, D), lambda i: (i, 0, 0)),
        out_shape=jax.ShapeDtypeStruct((B, L, D), jnp.float32),
    )(t, v, g, tw1c, tb1c, tw2t, vw1c, vw2t, eye)


def kernel(x, type_table, time_w1, time_b1, time_w2, val_w1, val_b1, val_w2):
    pad = ((0, 0), (0, LP - L))
    idx = jnp.pad(x[..., 0], pad).astype(jnp.int32).reshape(NP)
    t = jnp.pad(x[..., 1], pad).reshape(GRID, 1, TOKB)
    v = jnp.pad(x[..., 2], pad).reshape(GRID, 1, TOKB)

    tbl_i32 = lax.bitcast_convert_type(
        type_table.astype(jnp.bfloat16).reshape(V, DH, 2),
        jnp.int32)
    tbl_i32 = jnp.pad(tbl_i32, ((0, 0), (0, 1))).reshape(V * DHP)
    g = _sc_gather(tbl_i32, idx)

    tw1c = jnp.transpose(time_w1)
    tb1c = time_b1.reshape(D, 1)
    vw1c = jnp.transpose(val_w1)
    tw2t = jnp.transpose(time_w2).astype(jnp.bfloat16)
    vw2t = jnp.transpose(val_w2).astype(jnp.bfloat16)
    eye = jnp.eye(D, dtype=jnp.bfloat16)

    return _tc_dense_add(t, v, g, tw1c, tb1c, tw2t, vw1c, vw2t, eye)


# speedup vs baseline: 2.1473x; 2.1473x over previous
import functools

import jax
import jax.numpy as jnp
from jax import lax
from jax.experimental import pallas as pl
from jax.experimental.pallas import tpu as pltpu
from jax.experimental.pallas import tpu_sc as plsc

B, L, D, V = 4096, 50, 128, 1000
LP = 56
NP = B * LP
NC, NS = 2, 16
NW = NC * NS
BPW = NP // NW
CHUNK = 112
NCHUNK = BPW // CHUNK

TOKB = 1792
BB = TOKB // LP
GRID = NP // TOKB


DH = D // 2


def _sc_gather(tbl_i32, idx):
    mesh = plsc.VectorSubcoreMesh(core_axis_name="c", subcore_axis_name="s")

    @functools.partial(
        pl.kernel,
        mesh=mesh,
        out_type=jax.ShapeDtypeStruct((NP, D), jnp.float32),
        compiler_params=pltpu.CompilerParams(needs_layout_passes=False),
        scratch_types=[
            pltpu.VMEM((V * DH,), jnp.int32),
            pltpu.VMEM((BPW,), jnp.int32),
            pltpu.VMEM((2, CHUNK, D), jnp.float32),
            pltpu.SemaphoreType.DMA,
            pltpu.SemaphoreType.DMA,
        ],
    )
    def gather_kernel(tbl_hbm, idx_hbm, out_hbm, tbl_v, idx_all, stage, s0, s1):
        sem_w = (s0, s1)
        wid = lax.axis_index("s") * NC + lax.axis_index("c")
        base = wid * BPW
        pltpu.sync_copy(tbl_hbm, tbl_v)
        pltpu.sync_copy(idx_hbm.at[pl.ds(base, BPW)], idx_all)
        lanes = lax.iota(jnp.int32, 16)
        qvecs = [lanes + 16 * q for q in range(4)]

        def w_copy(c, b):
            off = pl.multiple_of(base + c * CHUNK, 8)
            return pltpu.make_async_copy(
                stage.at[b], out_hbm.at[pl.ds(off, CHUNK)], sem_w[b])

        def chunk(c, b):
            @pl.when(c >= 2)
            def _():
                w_copy(c - 2, b).wait()

            def group(tg, carry):
                iv = idx_all[pl.ds(c * CHUNK + tg * 16, 16)]
                ivb = iv * DH

                @plsc.parallel_loop(0, 16, unroll=8)
                def jbody(j):
                    sel = jnp.broadcast_to(j, (16,)).astype(jnp.int32)
                    rb = jnp.take_along_axis(ivb, sel, axis=0)
                    tok = tg * 16 + j
                    for q in range(4):
                        w = plsc.load_gather(tbl_v, [rb + qvecs[q]])
                        lo = plsc.bitcast(jnp.left_shift(w, 16), jnp.float32)
                        hi = plsc.bitcast(
                            jnp.bitwise_and(w, jnp.int32(-65536)), jnp.float32)
                        stage[b, tok, pl.ds(q * 16, 16)] = lo
                        stage[b, tok, pl.ds(DH + q * 16, 16)] = hi

                return carry

            lax.fori_loop(0, CHUNK // 16, group, 0)
            w_copy(c, b).start()

        def outer(c2, carry):
            for b in range(2):
                chunk(c2 * 2 + b, b)
            return carry

        lax.fori_loop(0, NCHUNK // 2, outer, 0)
        w_copy(NCHUNK - 2, 0).wait()
        w_copy(NCHUNK - 1, 1).wait()

    return gather_kernel(tbl_i32, idx)


def _tc_body(t_ref, v_ref, g_ref, tw1_ref, tb1_ref, tw2t_ref, vw1_ref,
             vw2t_ref, eye_ref, out_ref):
    tb = jnp.broadcast_to(t_ref[0], (D, TOKB))
    vb = jnp.broadcast_to(v_ref[0], (D, TOKB))
    at = jnp.tanh(tw1_ref[...] * tb + tb1_ref[...])
    bt = jnp.tanh(vw1_ref[...] * vb)
    st = lax.dot_general(tw2t_ref[...], at.astype(jnp.bfloat16),
                         (((1,), (0,)), ((), ())),
                         preferred_element_type=jnp.float32)
    st = st + lax.dot_general(vw2t_ref[...], bt.astype(jnp.bfloat16),
                              (((1,), (0,)), ((), ())),
                              preferred_element_type=jnp.float32)
    s = lax.dot_general(st.astype(jnp.bfloat16), eye_ref[...],
                        (((0,), (0,)), ((), ())),
                        preferred_element_type=jnp.float32)
    r = jnp.reshape(g_ref[...] + s, (BB, LP, D))
    out_ref[...] = r[:, :L, :]


def _tc_dense_add(t, v, g, tw1c, tb1c, tw2t, vw1c, vw2t, eye):
    wspec = pl.BlockSpec((D, 1), lambda i: (0, 0))
    mspec = pl.BlockSpec((D, D), lambda i: (0, 0))
    return pl.pallas_call(
        _tc_body,
        grid=(GRID,),
        in_specs=[
            pl.BlockSpec((1, 1, TOKB), lambda i: (i, 0, 0)),
            pl.BlockSpec((1, 1, TOKB), lambda i: (i, 0, 0)),
            pl.BlockSpec((TOKB, D), lambda i: (i, 0)),
            wspec, wspec, mspec, wspec, mspec, mspec,
        ],
        out_specs=pl.BlockSpec((BB, L, D), lambda i: (i, 0, 0)),
        out_shape=jax.ShapeDtypeStruct((B, L, D), jnp.float32),
    )(t, v, g, tw1c, tb1c, tw2t, vw1c, vw2t, eye)


def kernel(x, type_table, time_w1, time_b1, time_w2, val_w1, val_b1, val_w2):
    pad = ((0, 0), (0, LP - L))
    idx = jnp.pad(x[..., 0], pad).astype(jnp.int32).reshape(NP)
    t = jnp.pad(x[..., 1], pad).reshape(GRID, 1, TOKB)
    v = jnp.pad(x[..., 2], pad).reshape(GRID, 1, TOKB)

    tb16 = type_table.astype(jnp.bfloat16)
    tbl_i32 = lax.bitcast_convert_type(
        jnp.stack([tb16[:, :DH], tb16[:, DH:]], axis=-1),
        jnp.int32).reshape(V * DH)
    g = _sc_gather(tbl_i32, idx)

    tw1c = jnp.transpose(time_w1)
    tb1c = time_b1.reshape(D, 1)
    vw1c = jnp.transpose(val_w1)
    tw2t = jnp.transpose(time_w2).astype(jnp.bfloat16)
    vw2t = jnp.transpose(val_w2).astype(jnp.bfloat16)
    eye = jnp.eye(D, dtype=jnp.bfloat16)

    return _tc_dense_add(t, v, g, tw1c, tb1c, tw2t, vw1c, vw2t, eye)


# speedup vs baseline: 2.5727x; 1.1981x over previous
import functools

import jax
import jax.numpy as jnp
from jax import lax
from jax.experimental import pallas as pl
from jax.experimental.pallas import tpu as pltpu
from jax.experimental.pallas import tpu_sc as plsc

B, L, D, V = 4096, 50, 128, 1000
LP = 56
NP = B * LP
NC, NS = 2, 16
NW = NC * NS
BPW = NP // NW
CHUNK = 112
NCHUNK = BPW // CHUNK

TOKB = 3584
BB = TOKB // LP
GRID = NP // TOKB


DH = D // 2


def _sc_gather(tbl_i32, idx):
    mesh = plsc.VectorSubcoreMesh(core_axis_name="c", subcore_axis_name="s")

    @functools.partial(
        pl.kernel,
        mesh=mesh,
        out_type=jax.ShapeDtypeStruct((NP, D), jnp.float32),
        compiler_params=pltpu.CompilerParams(needs_layout_passes=False),
        scratch_types=[
            pltpu.VMEM((V * DH,), jnp.int32),
            pltpu.VMEM((BPW,), jnp.int32),
            pltpu.VMEM((2, CHUNK, D), jnp.float32),
            pltpu.SemaphoreType.DMA,
            pltpu.SemaphoreType.DMA,
        ],
    )
    def gather_kernel(tbl_hbm, idx_hbm, out_hbm, tbl_v, idx_all, stage, s0, s1):
        sem_w = (s0, s1)
        wid = lax.axis_index("s") * NC + lax.axis_index("c")
        base = wid * BPW
        pltpu.sync_copy(tbl_hbm, tbl_v)
        pltpu.sync_copy(idx_hbm.at[pl.ds(base, BPW)], idx_all)
        lanes = lax.iota(jnp.int32, 16)
        qvecs = [lanes + 16 * q for q in range(4)]

        def w_copy(c, b):
            off = pl.multiple_of(base + c * CHUNK, 8)
            return pltpu.make_async_copy(
                stage.at[b], out_hbm.at[pl.ds(off, CHUNK)], sem_w[b])

        def chunk(c, b):
            @pl.when(c >= 2)
            def _():
                w_copy(c - 2, b).wait()

            def group(tg, carry):
                iv = idx_all[pl.ds(c * CHUNK + tg * 16, 16)]
                ivb = iv * DH

                @plsc.parallel_loop(0, 16, unroll=8)
                def jbody(j):
                    sel = jnp.broadcast_to(j, (16,)).astype(jnp.int32)
                    rb = jnp.take_along_axis(ivb, sel, axis=0)
                    tok = tg * 16 + j
                    for q in range(4):
                        w = plsc.load_gather(tbl_v, [rb + qvecs[q]])
                        lo = plsc.bitcast(jnp.left_shift(w, 16), jnp.float32)
                        hi = plsc.bitcast(
                            jnp.bitwise_and(w, jnp.int32(-65536)), jnp.float32)
                        stage[b, tok, pl.ds(q * 16, 16)] = lo
                        stage[b, tok, pl.ds(DH + q * 16, 16)] = hi

                return carry

            lax.fori_loop(0, CHUNK // 16, group, 0)
            w_copy(c, b).start()

        def outer(c2, carry):
            for b in range(2):
                chunk(c2 * 2 + b, b)
            return carry

        lax.fori_loop(0, NCHUNK // 2, outer, 0)
        w_copy(NCHUNK - 2, 0).wait()
        w_copy(NCHUNK - 1, 1).wait()

    return gather_kernel(tbl_i32, idx)


def _tc_body(t_ref, v_ref, g_ref, tw1_ref, tb1_ref, tw2t_ref, vw1_ref,
             vw2t_ref, eye_ref, out_ref):
    tb = jnp.broadcast_to(t_ref[0], (D, TOKB))
    vb = jnp.broadcast_to(v_ref[0], (D, TOKB))
    at = jnp.tanh(tw1_ref[...] * tb + tb1_ref[...])
    bt = jnp.tanh(vw1_ref[...] * vb)
    st = lax.dot_general(tw2t_ref[...], at.astype(jnp.bfloat16),
                         (((1,), (0,)), ((), ())),
                         preferred_element_type=jnp.float32)
    st = st + lax.dot_general(vw2t_ref[...], bt.astype(jnp.bfloat16),
                              (((1,), (0,)), ((), ())),
                              preferred_element_type=jnp.float32)
    s = lax.dot_general(st.astype(jnp.bfloat16), eye_ref[...],
                        (((0,), (0,)), ((), ())),
                        preferred_element_type=jnp.float32)
    r = jnp.reshape(g_ref[...] + s, (BB, LP, D))
    out_ref[...] = r[:, :L, :]


def _tc_dense_add(t, v, g, tw1c, tb1c, tw2t, vw1c, vw2t, eye):
    wspec = pl.BlockSpec((D, 1), lambda i: (0, 0))
    mspec = pl.BlockSpec((D, D), lambda i: (0, 0))
    return pl.pallas_call(
        _tc_body,
        grid=(GRID,),
        in_specs=[
            pl.BlockSpec((1, 1, TOKB), lambda i: (i, 0, 0)),
            pl.BlockSpec((1, 1, TOKB), lambda i: (i, 0, 0)),
            pl.BlockSpec((TOKB, D), lambda i: (i, 0)),
            wspec, wspec, mspec, wspec, mspec, mspec,
        ],
        out_specs=pl.BlockSpec((BB, L, D), lambda i: (i, 0, 0)),
        out_shape=jax.ShapeDtypeStruct((B, L, D), jnp.float32),
    )(t, v, g, tw1c, tb1c, tw2t, vw1c, vw2t, eye)


def kernel(x, type_table, time_w1, time_b1, time_w2, val_w1, val_b1, val_w2):
    pad = ((0, 0), (0, LP - L))
    idx = jnp.pad(x[..., 0], pad).astype(jnp.int32).reshape(NP)
    t = jnp.pad(x[..., 1], pad).reshape(GRID, 1, TOKB)
    v = jnp.pad(x[..., 2], pad).reshape(GRID, 1, TOKB)

    tb16 = type_table.astype(jnp.bfloat16)
    tbl_i32 = lax.bitcast_convert_type(
        jnp.stack([tb16[:, :DH], tb16[:, DH:]], axis=-1),
        jnp.int32).reshape(V * DH)
    g = _sc_gather(tbl_i32, idx)

    tw1c = jnp.transpose(time_w1)
    tb1c = time_b1.reshape(D, 1)
    vw1c = jnp.transpose(val_w1)
    tw2t = jnp.transpose(time_w2).astype(jnp.bfloat16)
    vw2t = jnp.transpose(val_w2).astype(jnp.bfloat16)
    eye = jnp.eye(D, dtype=jnp.bfloat16)

    return _tc_dense_add(t, v, g, tw1c, tb1c, tw2t, vw1c, vw2t, eye)


# speedup vs baseline: 2.8022x; 1.0892x over previous
import functools

import jax
import jax.numpy as jnp
from jax import lax
from jax.experimental import pallas as pl
from jax.experimental.pallas import tpu as pltpu
from jax.experimental.pallas import tpu_sc as plsc

B, L, D, V = 4096, 50, 128, 1000
LP = 56
NP = B * LP
NC, NS = 2, 16
NW = NC * NS
BPW = NP // NW
CHUNK = 112
NCHUNK = BPW // CHUNK

TOKB = 7168
BB = TOKB // LP
GRID = NP // TOKB


DH = D // 2


def _sc_gather(tbl_i32, idx):
    mesh = plsc.VectorSubcoreMesh(core_axis_name="c", subcore_axis_name="s")

    @functools.partial(
        pl.kernel,
        mesh=mesh,
        out_type=jax.ShapeDtypeStruct((NP, D), jnp.float32),
        compiler_params=pltpu.CompilerParams(needs_layout_passes=False),
        scratch_types=[
            pltpu.VMEM((V * DH,), jnp.int32),
            pltpu.VMEM((BPW,), jnp.int32),
            pltpu.VMEM((2, CHUNK, D), jnp.float32),
            pltpu.SemaphoreType.DMA,
            pltpu.SemaphoreType.DMA,
        ],
    )
    def gather_kernel(tbl_hbm, idx_hbm, out_hbm, tbl_v, idx_all, stage, s0, s1):
        sem_w = (s0, s1)
        wid = lax.axis_index("s") * NC + lax.axis_index("c")
        base = wid * BPW
        pltpu.sync_copy(tbl_hbm, tbl_v)
        pltpu.sync_copy(idx_hbm.at[pl.ds(base, BPW)], idx_all)
        lanes = lax.iota(jnp.int32, 16)
        qvecs = [lanes + 16 * q for q in range(4)]

        def w_copy(c, b):
            off = pl.multiple_of(base + c * CHUNK, 8)
            return pltpu.make_async_copy(
                stage.at[b], out_hbm.at[pl.ds(off, CHUNK)], sem_w[b])

        def chunk(c, b):
            @pl.when(c >= 2)
            def _():
                w_copy(c - 2, b).wait()

            def group(tg, carry):
                iv = idx_all[pl.ds(c * CHUNK + tg * 16, 16)]
                ivb = iv * DH

                @plsc.parallel_loop(0, 16, unroll=8)
                def jbody(j):
                    sel = jnp.broadcast_to(j, (16,)).astype(jnp.int32)
                    rb = jnp.take_along_axis(ivb, sel, axis=0)
                    tok = tg * 16 + j
                    for q in range(4):
                        w = plsc.load_gather(tbl_v, [rb + qvecs[q]])
                        lo = plsc.bitcast(jnp.left_shift(w, 16), jnp.float32)
                        hi = plsc.bitcast(
                            jnp.bitwise_and(w, jnp.int32(-65536)), jnp.float32)
                        stage[b, tok, pl.ds(q * 16, 16)] = lo
                        stage[b, tok, pl.ds(DH + q * 16, 16)] = hi

                return carry

            lax.fori_loop(0, CHUNK // 16, group, 0)
            w_copy(c, b).start()

        def outer(c2, carry):
            for b in range(2):
                chunk(c2 * 2 + b, b)
            return carry

        lax.fori_loop(0, NCHUNK // 2, outer, 0)
        w_copy(NCHUNK - 2, 0).wait()
        w_copy(NCHUNK - 1, 1).wait()

    return gather_kernel(tbl_i32, idx)


def _tc_body(t_ref, v_ref, g_ref, tw1_ref, tb1_ref, tw2t_ref, vw1_ref,
             vw2t_ref, eye_ref, out_ref):
    tb = jnp.broadcast_to(t_ref[0], (D, TOKB))
    vb = jnp.broadcast_to(v_ref[0], (D, TOKB))
    at = jnp.tanh(tw1_ref[...] * tb + tb1_ref[...])
    bt = jnp.tanh(vw1_ref[...] * vb)
    st = lax.dot_general(tw2t_ref[...], at.astype(jnp.bfloat16),
                         (((1,), (0,)), ((), ())),
                         preferred_element_type=jnp.float32)
    st = st + lax.dot_general(vw2t_ref[...], bt.astype(jnp.bfloat16),
                              (((1,), (0,)), ((), ())),
                              preferred_element_type=jnp.float32)
    s = lax.dot_general(st.astype(jnp.bfloat16), eye_ref[...],
                        (((0,), (0,)), ((), ())),
                        preferred_element_type=jnp.float32)
    r = jnp.reshape(g_ref[...] + s, (BB, LP, D))
    out_ref[...] = r[:, :L, :]


def _tc_dense_add(t, v, g, tw1c, tb1c, tw2t, vw1c, vw2t, eye):
    wspec = pl.BlockSpec((D, 1), lambda i: (0, 0))
    mspec = pl.BlockSpec((D, D), lambda i: (0, 0))
    return pl.pallas_call(
        _tc_body,
        grid=(GRID,),
        in_specs=[
            pl.BlockSpec((1, 1, TOKB), lambda i: (i, 0, 0)),
            pl.BlockSpec((1, 1, TOKB), lambda i: (i, 0, 0)),
            pl.BlockSpec((TOKB, D), lambda i: (i, 0)),
            wspec, wspec, mspec, wspec, mspec, mspec,
        ],
        out_specs=pl.BlockSpec((BB, L, D), lambda i: (i, 0, 0)),
        out_shape=jax.ShapeDtypeStruct((B, L, D), jnp.float32),
    )(t, v, g, tw1c, tb1c, tw2t, vw1c, vw2t, eye)


def kernel(x, type_table, time_w1, time_b1, time_w2, val_w1, val_b1, val_w2):
    pad = ((0, 0), (0, LP - L))
    idx = jnp.pad(x[..., 0], pad).astype(jnp.int32).reshape(NP)
    t = jnp.pad(x[..., 1], pad).reshape(GRID, 1, TOKB)
    v = jnp.pad(x[..., 2], pad).reshape(GRID, 1, TOKB)

    tb16 = type_table.astype(jnp.bfloat16)
    tbl_i32 = lax.bitcast_convert_type(
        jnp.stack([tb16[:, :DH], tb16[:, DH:]], axis=-1),
        jnp.int32).reshape(V * DH)
    g = _sc_gather(tbl_i32, idx)

    tw1c = jnp.transpose(time_w1)
    tb1c = time_b1.reshape(D, 1)
    vw1c = jnp.transpose(val_w1)
    tw2t = jnp.transpose(time_w2).astype(jnp.bfloat16)
    vw2t = jnp.transpose(val_w2).astype(jnp.bfloat16)
    eye = jnp.eye(D, dtype=jnp.bfloat16)

    return _tc_dense_add(t, v, g, tw1c, tb1c, tw2t, vw1c, vw2t, eye)


# speedup vs baseline: 2.8819x; 1.0284x over previous
import functools

import jax
import jax.numpy as jnp
from jax import lax
from jax.experimental import pallas as pl
from jax.experimental.pallas import tpu as pltpu
from jax.experimental.pallas import tpu_sc as plsc

B, L, D, V = 4096, 50, 128, 1000
LP = 56
NP = B * LP
NC, NS = 2, 16
NW = NC * NS
BPW = NP // NW
CHUNK = 112
NCHUNK = BPW // CHUNK

TOKB = 14336
BB = TOKB // LP
GRID = NP // TOKB


DH = D // 2


def _sc_gather(tbl_i32, idx):
    mesh = plsc.VectorSubcoreMesh(core_axis_name="c", subcore_axis_name="s")

    @functools.partial(
        pl.kernel,
        mesh=mesh,
        out_type=jax.ShapeDtypeStruct((NP, D), jnp.float32),
        compiler_params=pltpu.CompilerParams(needs_layout_passes=False),
        scratch_types=[
            pltpu.VMEM((V * DH,), jnp.int32),
            pltpu.VMEM((BPW,), jnp.int32),
            pltpu.VMEM((2, CHUNK, D), jnp.float32),
            pltpu.SemaphoreType.DMA,
            pltpu.SemaphoreType.DMA,
        ],
    )
    def gather_kernel(tbl_hbm, idx_hbm, out_hbm, tbl_v, idx_all, stage, s0, s1):
        sem_w = (s0, s1)
        wid = lax.axis_index("s") * NC + lax.axis_index("c")
        base = wid * BPW
        pltpu.sync_copy(tbl_hbm, tbl_v)
        pltpu.sync_copy(idx_hbm.at[pl.ds(base, BPW)], idx_all)
        lanes = lax.iota(jnp.int32, 16)
        qvecs = [lanes + 16 * q for q in range(4)]

        def w_copy(c, b):
            off = pl.multiple_of(base + c * CHUNK, 8)
            return pltpu.make_async_copy(
                stage.at[b], out_hbm.at[pl.ds(off, CHUNK)], sem_w[b])

        def chunk(c, b):
            @pl.when(c >= 2)
            def _():
                w_copy(c - 2, b).wait()

            def group(tg, carry):
                iv = idx_all[pl.ds(c * CHUNK + tg * 16, 16)]
                ivb = iv * DH

                @plsc.parallel_loop(0, 16, unroll=8)
                def jbody(j):
                    sel = jnp.broadcast_to(j, (16,)).astype(jnp.int32)
                    rb = jnp.take_along_axis(ivb, sel, axis=0)
                    tok = tg * 16 + j
                    for q in range(4):
                        w = plsc.load_gather(tbl_v, [rb + qvecs[q]])
                        lo = plsc.bitcast(jnp.left_shift(w, 16), jnp.float32)
                        hi = plsc.bitcast(
                            jnp.bitwise_and(w, jnp.int32(-65536)), jnp.float32)
                        stage[b, tok, pl.ds(q * 16, 16)] = lo
                        stage[b, tok, pl.ds(DH + q * 16, 16)] = hi

                return carry

            lax.fori_loop(0, CHUNK // 16, group, 0)
            w_copy(c, b).start()

        def outer(c2, carry):
            for b in range(2):
                chunk(c2 * 2 + b, b)
            return carry

        lax.fori_loop(0, NCHUNK // 2, outer, 0)
        w_copy(NCHUNK - 2, 0).wait()
        w_copy(NCHUNK - 1, 1).wait()

    return gather_kernel(tbl_i32, idx)


def _tc_body(t_ref, v_ref, g_ref, tw1_ref, tb1_ref, tw2t_ref, vw1_ref,
             vw2t_ref, eye_ref, out_ref):
    tb = jnp.broadcast_to(t_ref[0], (D, TOKB))
    vb = jnp.broadcast_to(v_ref[0], (D, TOKB))
    at = jnp.tanh(tw1_ref[...] * tb + tb1_ref[...])
    bt = jnp.tanh(vw1_ref[...] * vb)
    st = lax.dot_general(tw2t_ref[...], at.astype(jnp.bfloat16),
                         (((1,), (0,)), ((), ())),
                         preferred_element_type=jnp.float32)
    st = st + lax.dot_general(vw2t_ref[...], bt.astype(jnp.bfloat16),
                              (((1,), (0,)), ((), ())),
                              preferred_element_type=jnp.float32)
    s = lax.dot_general(st.astype(jnp.bfloat16), eye_ref[...],
                        (((0,), (0,)), ((), ())),
                        preferred_element_type=jnp.float32)
    r = jnp.reshape(g_ref[...] + s, (BB, LP, D))
    out_ref[...] = r[:, :L, :]


def _tc_dense_add(t, v, g, tw1c, tb1c, tw2t, vw1c, vw2t, eye):
    wspec = pl.BlockSpec((D, 1), lambda i: (0, 0))
    mspec = pl.BlockSpec((D, D), lambda i: (0, 0))
    return pl.pallas_call(
        _tc_body,
        grid=(GRID,),
        in_specs=[
            pl.BlockSpec((1, 1, TOKB), lambda i: (i, 0, 0)),
            pl.BlockSpec((1, 1, TOKB), lambda i: (i, 0, 0)),
            pl.BlockSpec((TOKB, D), lambda i: (i, 0)),
            wspec, wspec, mspec, wspec, mspec, mspec,
        ],
        out_specs=pl.BlockSpec((BB, L, D), lambda i: (i, 0, 0)),
        out_shape=jax.ShapeDtypeStruct((B, L, D), jnp.float32),
    )(t, v, g, tw1c, tb1c, tw2t, vw1c, vw2t, eye)


def kernel(x, type_table, time_w1, time_b1, time_w2, val_w1, val_b1, val_w2):
    pad = ((0, 0), (0, LP - L))
    idx = jnp.pad(x[..., 0], pad).astype(jnp.int32).reshape(NP)
    t = jnp.pad(x[..., 1], pad).reshape(GRID, 1, TOKB)
    v = jnp.pad(x[..., 2], pad).reshape(GRID, 1, TOKB)

    tb16 = type_table.astype(jnp.bfloat16)
    tbl_i32 = lax.bitcast_convert_type(
        jnp.stack([tb16[:, :DH], tb16[:, DH:]], axis=-1),
        jnp.int32).reshape(V * DH)
    g = _sc_gather(tbl_i32, idx)

    tw1c = jnp.transpose(time_w1)
    tb1c = time_b1.reshape(D, 1)
    vw1c = jnp.transpose(val_w1)
    tw2t = jnp.transpose(time_w2).astype(jnp.bfloat16)
    vw2t = jnp.transpose(val_w2).astype(jnp.bfloat16)
    eye = jnp.eye(D, dtype=jnp.bfloat16)

    return _tc_dense_add(t, v, g, tw1c, tb1c, tw2t, vw1c, vw2t, eye)
